# Initial kernel scaffold; baseline (speedup 1.0000x reference)
import jax
import jax.numpy as jnp
from jax.experimental import pallas as pl


def kernel(x):
    raise NotImplementedError("write your pallas kernel here")


# speedup vs baseline: 2.1534x; 2.1534x over previous
import functools

import jax
import jax.numpy as jnp
from jax.experimental import pallas as pl
from jax.experimental.pallas import tpu as pltpu

N, D, K = 8192, 256, 1024
ITERS = 5
BM = 512
NBLK = N // BM


def _assign_body(x_ref, c_ref, labels_ref):
    x = x_ref[...]
    c = c_ref[...]
    x2 = jnp.sum(x * x, axis=1, keepdims=True)
    c2 = jnp.sum(c * c, axis=1)[None, :]
    d2 = x2 - 2.0 * jnp.dot(x, c.T) + c2
    labels_ref[0, 0, :] = jnp.argmin(d2, axis=1).astype(jnp.int32)


def _assign(x, c):
    return pl.pallas_call(
        _assign_body,
        grid=(NBLK,),
        in_specs=[
            pl.BlockSpec((BM, D), lambda i: (i, 0)),
            pl.BlockSpec((K, D), lambda i: (0, 0)),
        ],
        out_specs=pl.BlockSpec((1, 1, BM), lambda i: (i, 0, 0)),
        out_shape=jax.ShapeDtypeStruct((NBLK, 1, BM), jnp.int32),
    )(x, c)


def _segsum_body(x_ref, labels_ref, sums_ref, counts_ref):
    i = pl.program_id(0)

    @pl.when(i == 0)
    def _():
        sums_ref[...] = jnp.zeros_like(sums_ref)
        counts_ref[...] = jnp.zeros_like(counts_ref)

    x = x_ref[...]
    lbl = labels_ref[0, 0, :]
    onehot = (lbl[:, None] == jax.lax.broadcasted_iota(
        jnp.int32, (BM, K), 1)).astype(jnp.float32)
    sums_ref[...] += jax.lax.dot_general(
        onehot, x, (((0,), (0,)), ((), ())),
        preferred_element_type=jnp.float32,
        precision=jax.lax.Precision.HIGHEST)
    counts_ref[0, :] += jnp.sum(onehot, axis=0)


def _segsum(x, labels):
    return pl.pallas_call(
        _segsum_body,
        grid=(NBLK,),
        in_specs=[
            pl.BlockSpec((BM, D), lambda i: (i, 0)),
            pl.BlockSpec((1, 1, BM), lambda i: (i, 0, 0)),
        ],
        out_specs=[
            pl.BlockSpec((K, D), lambda i: (0, 0)),
            pl.BlockSpec((1, K), lambda i: (0, 0)),
        ],
        out_shape=[
            jax.ShapeDtypeStruct((K, D), jnp.float32),
            jax.ShapeDtypeStruct((1, K), jnp.float32),
        ],
    )(x, labels)


def _update_body(sums_ref, counts_ref, c_ref, out_ref):
    sums = sums_ref[...]
    counts = counts_ref[...]
    c = c_ref[...]
    new_c = sums / jnp.maximum(counts, 1.0)
    out_ref[...] = jnp.where(counts > 0, new_c, c)


def _update(sums, counts, c):
    return pl.pallas_call(
        _update_body,
        in_specs=[
            pl.BlockSpec((K, D), lambda: (0, 0)),
            pl.BlockSpec((K, 1), lambda: (0, 0)),
            pl.BlockSpec((K, D), lambda: (0, 0)),
        ],
        out_specs=pl.BlockSpec((K, D), lambda: (0, 0)),
        out_shape=jax.ShapeDtypeStruct((K, D), jnp.float32),
    )(sums, counts, c)


def kernel(x):
    x = x.reshape(x.shape[0], -1)
    c = x[:K]
    for _ in range(ITERS):
        labels = _assign(x, c)
        sums, counts = _segsum(x, labels)
        c = _update(sums, counts.reshape(K, 1), c)
    labels = _assign(x, c)
    return labels.reshape(N)
